# baseline (speedup 1.0000x reference)
import functools

import jax
import jax.numpy as jnp
from jax import lax
from jax.experimental import pallas as pl
from jax.experimental.pallas import tpu as pltpu
from jax.experimental.pallas import tpu_sc as plsc

N_TOK = 16384
D = 2048
E = 64
K = 8
BT = 2048

NC = 2
NS = 16
NW = NC * NS
TPW = N_TOK // NW
L = 16
GRP = TPW // L
NEG = -3.0e38


def _matmul_block(x_ref, w_ref, logits_ref):
    logits_ref[...] = jax.lax.dot_general(
        x_ref[...], w_ref[...], (((1,), (1,)), ((), ())),
        preferred_element_type=jnp.float32,
        precision=jax.lax.Precision.DEFAULT,
    )


def _tc_logits(hidden_states, gate_weight):
    return pl.pallas_call(
        _matmul_block,
        grid=(N_TOK // BT,),
        in_specs=[
            pl.BlockSpec((BT, D), lambda i: (i, 0)),
            pl.BlockSpec((E, D), lambda i: (0, 0)),
        ],
        out_specs=pl.BlockSpec((BT, E), lambda i: (i, 0)),
        out_shape=jax.ShapeDtypeStruct((N_TOK, E), jnp.float32),
    )(hidden_states, gate_weight)


def _sc_router_body(logits_hbm, idx_hbm, gates_hbm, buf, idxb, gateb):
    c = lax.axis_index("c")
    s = lax.axis_index("s")
    wid = s * NC + c
    base = wid * TPW
    pltpu.sync_copy(logits_hbm.at[pl.ds(base * E, TPW * E)], buf)

    lane = lax.broadcasted_iota(jnp.int32, (L,), 0)

    def group_body(g, _):
        rows_e = (g * L + lane) * E
        rows_k = (g * L + lane) * K

        def expert_body(e, carry):
            vs = list(carry[:K])
            ids = list(carry[K:])
            val = plsc.load_gather(buf, [rows_e + e])
            vid = jnp.full((L,), 0, jnp.int32) + e
            for j in range(K):
                swap = val > vs[j]
                nv = jnp.where(swap, val, vs[j])
                val = jnp.where(swap, vs[j], val)
                ni = jnp.where(swap, vid, ids[j])
                vid = jnp.where(swap, ids[j], vid)
                vs[j] = nv
                ids[j] = ni
            return tuple(vs) + tuple(ids)

        init = tuple(jnp.full((L,), NEG, jnp.float32) for _ in range(K)) + \
               tuple(jnp.full((L,), 0, jnp.int32) for _ in range(K))
        res = lax.fori_loop(0, E, expert_body, init)
        vs = res[:K]
        ids = res[K:]
        exps = [jnp.exp(v - vs[0]) for v in vs]
        tot = exps[0]
        for t in exps[1:]:
            tot = tot + t
        for j in range(K):
            plsc.store_scatter(idxb, [rows_k + j], ids[j])
            plsc.store_scatter(gateb, [rows_k + j], exps[j] / tot)
        return 0

    lax.fori_loop(0, GRP, group_body, 0)
    pltpu.sync_copy(idxb, idx_hbm.at[pl.ds(base * K, TPW * K)])
    pltpu.sync_copy(gateb, gates_hbm.at[pl.ds(base * K, TPW * K)])


def _sc_router(logits_flat):
    mesh = plsc.VectorSubcoreMesh(core_axis_name="c", subcore_axis_name="s")
    return pl.kernel(
        _sc_router_body,
        out_type=[
            jax.ShapeDtypeStruct((N_TOK * K,), jnp.int32),
            jax.ShapeDtypeStruct((N_TOK * K,), jnp.float32),
        ],
        mesh=mesh,
        compiler_params=pltpu.CompilerParams(needs_layout_passes=False),
        scratch_types=[
            pltpu.VMEM((TPW * E,), jnp.float32),
            pltpu.VMEM((TPW * K,), jnp.int32),
            pltpu.VMEM((TPW * K,), jnp.float32),
        ],
    )(logits_flat)


@jax.jit
def kernel(hidden_states, gate_weight):
    logits = _tc_logits(hidden_states, gate_weight)
    idx, gates = _sc_router(logits.reshape(-1))
    return (idx.reshape(N_TOK, K), gates.reshape(N_TOK, K), logits)
